# baseline (speedup 1.0000x reference)
import functools

import jax
import jax.numpy as jnp
from jax.experimental import pallas as pl
from jax.experimental.pallas import tpu as pltpu


def _se_block_kernel(x_ref, w1_ref, b1_ref, w2_ref, b2_ref, out_ref, y_ref,
                     *, inv_hw):
    x = x_ref[...]
    pooled = jnp.sum(x, axis=-1) * inv_hw
    h = jax.lax.dot_general(pooled, w1_ref[...],
                            (((1,), (1,)), ((), ())),
                            preferred_element_type=jnp.float32)
    h = jnp.maximum(h + b1_ref[...], 0.0)
    s = jax.lax.dot_general(h, w2_ref[...],
                            (((1,), (1,)), ((), ())),
                            preferred_element_type=jnp.float32)
    s = jax.nn.sigmoid(s + b2_ref[...])
    sb = s[:, :, None].astype(x.dtype)
    y_ref[...] = sb
    out_ref[...] = x * sb


def kernel(x, w1, b1, w2, b2):
    N, C, H, W = x.shape
    Cr = w1.shape[0]
    HW = H * W

    itemsize = jnp.dtype(x.dtype).itemsize
    B = 1
    for cand in (4, 2):
        if N % cand == 0 and 4 * cand * C * HW * itemsize <= 44 * 1024 * 1024:
            B = cand
            break

    x_flat = x.reshape(N, C, HW)
    w1f = w1.astype(jnp.float32)
    w2f = w2.astype(jnp.float32)
    b1f = b1.astype(jnp.float32).reshape(1, Cr)
    b2f = b2.astype(jnp.float32).reshape(1, C)

    out_flat, y3 = pl.pallas_call(
        functools.partial(_se_block_kernel, inv_hw=1.0 / HW),
        out_shape=(jax.ShapeDtypeStruct((N, C, HW), x.dtype),
                   jax.ShapeDtypeStruct((N, C, 1), x.dtype)),
        grid=(N // B,),
        in_specs=[
            pl.BlockSpec((B, C, HW), lambda n: (n, 0, 0)),
            pl.BlockSpec((Cr, C), lambda n: (0, 0)),
            pl.BlockSpec((1, Cr), lambda n: (0, 0)),
            pl.BlockSpec((C, Cr), lambda n: (0, 0)),
            pl.BlockSpec((1, C), lambda n: (0, 0)),
        ],
        out_specs=[
            pl.BlockSpec((B, C, HW), lambda n: (n, 0, 0)),
            pl.BlockSpec((B, C, 1), lambda n: (n, 0, 0)),
        ],
        compiler_params=pltpu.CompilerParams(
            dimension_semantics=("parallel",),
            vmem_limit_bytes=60 * 1024 * 1024),
    )(x_flat, w1f, b1f, w2f, b2f)

    return out_flat.reshape(N, C, H, W), y3.reshape(N, C, 1, 1)


# speedup vs baseline: 1.0017x; 1.0017x over previous
import functools

import jax
import jax.numpy as jnp
from jax.experimental import pallas as pl
from jax.experimental.pallas import tpu as pltpu


def _se_block_kernel(x_ref, w1_ref, b1_ref, w2_ref, b2_ref, out_ref, y_ref,
                     *, inv_hw):
    x = x_ref[...]
    pooled = jnp.sum(x, axis=-1) * inv_hw
    h = jax.lax.dot_general(pooled, w1_ref[...],
                            (((1,), (1,)), ((), ())),
                            preferred_element_type=jnp.float32)
    h = jnp.maximum(h + b1_ref[...], 0.0)
    s = jax.lax.dot_general(h, w2_ref[...],
                            (((1,), (1,)), ((), ())),
                            preferred_element_type=jnp.float32)
    s = jax.nn.sigmoid(s + b2_ref[...])
    sb = s[:, :, None].astype(x.dtype)
    y_ref[...] = sb
    out_ref[...] = x * sb


def kernel(x, w1, b1, w2, b2):
    N, C, H, W = x.shape
    Cr = w1.shape[0]
    HW = H * W

    itemsize = jnp.dtype(x.dtype).itemsize
    B = 1
    for cand in (4, 2):
        if N % cand == 0 and 4 * cand * C * HW * itemsize <= 44 * 1024 * 1024:
            B = cand
            break

    x_flat = x.reshape(N, C, HW)
    w1f = w1.astype(jnp.float32)
    w2f = w2.astype(jnp.float32)
    b1f = b1.astype(jnp.float32).reshape(1, Cr)
    b2f = b2.astype(jnp.float32).reshape(1, C)

    num_steps = N // B
    if num_steps % 2 == 0:
        half = num_steps // 2
        grid = (2, half)
        semantics = ("parallel", "arbitrary")
        big = lambda c, i: (c * half + i, 0, 0)
        const = lambda c, i: (0, 0)
    else:
        grid = (num_steps,)
        semantics = ("parallel",)
        big = lambda n: (n, 0, 0)
        const = lambda n: (0, 0)

    out_flat, y3 = pl.pallas_call(
        functools.partial(_se_block_kernel, inv_hw=1.0 / HW),
        out_shape=(jax.ShapeDtypeStruct((N, C, HW), x.dtype),
                   jax.ShapeDtypeStruct((N, C, 1), x.dtype)),
        grid=grid,
        in_specs=[
            pl.BlockSpec((B, C, HW), big),
            pl.BlockSpec((Cr, C), const),
            pl.BlockSpec((1, Cr), const),
            pl.BlockSpec((C, Cr), const),
            pl.BlockSpec((1, C), const),
        ],
        out_specs=[
            pl.BlockSpec((B, C, HW), big),
            pl.BlockSpec((B, C, 1), big),
        ],
        compiler_params=pltpu.CompilerParams(
            dimension_semantics=semantics,
            vmem_limit_bytes=60 * 1024 * 1024),
    )(x_flat, w1f, b1f, w2f, b2f)

    return out_flat.reshape(N, C, H, W), y3.reshape(N, C, 1, 1)
